# baseline (speedup 1.0000x reference)
import functools

import jax
import jax.numpy as jnp
from jax import lax
from jax.experimental import pallas as pl
from jax.experimental.pallas import tpu as pltpu
from jax.experimental.pallas import tpu_sc as plsc

N_NODES = 1000000
MEM_DIM = 64
BATCH = 16384
TIME_INIT = -1.0

_NUM_CORES = 2
_NUM_SUBCORES = 16
_NW = _NUM_CORES * _NUM_SUBCORES
_BPW = BATCH // _NW
_LANES = 16
_VPR = MEM_DIM // _LANES

_mesh = plsc.VectorSubcoreMesh(core_axis_name="c", subcore_axis_name="s")


@functools.partial(
    pl.kernel,
    mesh=_mesh,
    compiler_params=pltpu.CompilerParams(use_tc_tiling_on_sc=False),
    out_type=jax.ShapeDtypeStruct((BATCH, MEM_DIM), jnp.float32),
    scratch_types=[
        pltpu.VMEM((_BPW,), jnp.int32),
        pltpu.VMEM((_BPW, MEM_DIM), jnp.float32),
        pltpu.VMEM((_BPW,), jnp.float32),
        pltpu.VMEM((MEM_DIM,), jnp.float32),
        pltpu.SemaphoreType.DMA,
        pltpu.SemaphoreType.DMA,
    ],
)
def _gather_mem(mem_hbm, lu_hbm, dflt_hbm, idx_hbm, out_hbm,
                idx_v, rows_v, lu_v, dflt_v, sem_rows, sem_lu):
    wid = lax.axis_index("s") * _NUM_CORES + lax.axis_index("c")
    base = wid * _BPW

    pltpu.sync_copy(idx_hbm.at[pl.ds(base, _BPW)], idx_v)
    pltpu.sync_copy(dflt_hbm, dflt_v)

    cp_rows = pltpu.async_copy(mem_hbm.at[idx_v], rows_v, sem_rows)
    cp_lu = pltpu.async_copy(lu_hbm.at[idx_v], lu_v, sem_lu)
    cp_lu.wait()
    cp_rows.wait()

    dvecs = [dflt_v[pl.ds(j * _LANES, _LANES)] for j in range(_VPR)]

    def chunk_fix(c, carry):
        lu16 = lu_v[pl.ds(c * _LANES, _LANES)]
        for i in range(_LANES):
            @pl.when(lu16[i] == jnp.float32(TIME_INIT))
            def _():
                for j in range(_VPR):
                    rows_v[c * _LANES + i, pl.ds(j * _LANES, _LANES)] = dvecs[j]
        return carry

    lax.fori_loop(0, _BPW // _LANES, chunk_fix, 0)

    pltpu.sync_copy(rows_v, out_hbm.at[pl.ds(base, _BPW)])


def kernel(memory, last_update, default_memory, node_ids):
    idx = node_ids.astype(jnp.int32)
    return _gather_mem(memory, last_update, default_memory, idx)


# speedup vs baseline: 2.4971x; 2.4971x over previous
import functools

import jax
import jax.numpy as jnp
from jax import lax
from jax.experimental import pallas as pl
from jax.experimental.pallas import tpu as pltpu
from jax.experimental.pallas import tpu_sc as plsc

N_NODES = 1000000
MEM_DIM = 64
BATCH = 16384
TIME_INIT = -1.0

_NUM_CORES = 2
_NUM_SUBCORES = 16
_NW = _NUM_CORES * _NUM_SUBCORES
_BPW = BATCH // _NW
_LANES = 16
_VPR = MEM_DIM // _LANES
_CH = 64
_NCHUNK = _BPW // _CH

_mesh = plsc.VectorSubcoreMesh(core_axis_name="c", subcore_axis_name="s")


@functools.partial(
    pl.kernel,
    mesh=_mesh,
    out_type=jax.ShapeDtypeStruct((BATCH, MEM_DIM), jnp.float32),
    scratch_types=[
        pltpu.VMEM((_BPW,), jnp.int32),
        pltpu.VMEM((_BPW,), jnp.float32),
        pltpu.VMEM((MEM_DIM,), jnp.float32),
        pltpu.VMEM((_BPW, MEM_DIM), jnp.float32),
        pltpu.SemaphoreType.DMA,
        pltpu.SemaphoreType.DMA,
    ],
)
def _gather_mem(mem3_hbm, lu_hbm, dflt_hbm, idx_hbm, out_hbm,
                idx_v, lu_v, dflt_v, rows_v, sem_g, sem_lu):
    wid = lax.axis_index("s") * _NUM_CORES + lax.axis_index("c")
    base = wid * _BPW

    pltpu.sync_copy(idx_hbm.at[pl.ds(base, _BPW)], idx_v)
    pltpu.sync_copy(dflt_hbm, dflt_v)
    cp_lu = pltpu.async_copy(lu_hbm.at[idx_v], lu_v, sem_lu)

    def batch_body(c, carry):
        idxvecs = [idx_v[pl.ds(c * _CH + g * _LANES, _LANES)]
                   for g in range(_CH // _LANES)]
        cps = []
        for i in range(_CH):
            nid = idxvecs[i // _LANES][i % _LANES]
            t = lax.shift_right_logical(nid, 3)
            s = lax.bitwise_and(nid, 7)
            cps.append(pltpu.async_copy(
                mem3_hbm.at[t, s], rows_v.at[c * _CH + i], sem_g))
        for cp in cps:
            cp.wait()
        return carry

    lax.fori_loop(0, _NCHUNK, batch_body, 0)
    cp_lu.wait()

    dvecs = [dflt_v[pl.ds(j * _LANES, _LANES)] for j in range(_VPR)]

    def chunk_fix(c, carry):
        lu16 = lu_v[pl.ds(c * _LANES, _LANES)]
        for i in range(_LANES):
            @pl.when(lu16[i] == jnp.float32(TIME_INIT))
            def _():
                for j in range(_VPR):
                    rows_v[c * _LANES + i, pl.ds(j * _LANES, _LANES)] = dvecs[j]
        return carry

    lax.fori_loop(0, _BPW // _LANES, chunk_fix, 0)

    pltpu.sync_copy(rows_v, out_hbm.at[pl.ds(base, _BPW)])


def kernel(memory, last_update, default_memory, node_ids):
    idx = node_ids.astype(jnp.int32)
    mem3 = memory.reshape(N_NODES // 8, 8, MEM_DIM)
    return _gather_mem(mem3, last_update, default_memory, idx)
